# Initial kernel scaffold; baseline (speedup 1.0000x reference)
import jax
import jax.numpy as jnp
from jax.experimental import pallas as pl


def kernel(phw_list, filters):
    raise NotImplementedError("write your pallas kernel here")


# speedup vs baseline: 39.2840x; 39.2840x over previous
import functools

import jax
import jax.numpy as jnp
from jax import lax
from jax.experimental import pallas as pl
from jax.experimental.pallas import tpu as pltpu
from jax.experimental.pallas import tpu_sc as plsc

NFILT = 512
FH = FW = 64
H = W = 512
NPART = 8192
NC, NS, L = 2, 16, 16
NW = NC * NS
BAND = H // NW
FO = FH // 2


def _body(phw_hbm, filt_hbm, out_hbm, phw_v, hits_v, fbuf, band_v, sem):
    wid = lax.axis_index("s") * NC + lax.axis_index("c")
    y0 = (wid * BAND).astype(jnp.int32)
    lane = lax.iota(jnp.int32, L)
    zv = jnp.zeros((L,), jnp.float32)

    pltpu.sync_copy(phw_hbm, phw_v.at[pl.ds(0, NPART * 3)])

    def zero_row(i, carry):
        for s in range(W // L):
            band_v[i, s * L:(s + 1) * L] = zv
        return carry
    lax.fori_loop(0, BAND, zero_row, 0)

    def scan_chunk(k, cnt):
        base = k * L
        r = plsc.load_gather(phw_v, [(base + lane) * 3 + 1])
        hit = (r >= y0 - (FO - 1)) & (r <= y0 + BAND + (FO - 1))
        plsc.store_compressed(hits_v.at[pl.ds(cnt, L)], base + lane, mask=hit)
        return cnt + jnp.sum(hit.astype(jnp.int32))
    nhits = lax.fori_loop(0, NPART // L, scan_chunk, jnp.int32(0))

    def process(h, carry):
        pid = hits_v[pl.ds(h, L)][0]
        prc = phw_v[pl.ds(pid * 3, L)]
        p = prc[0]
        r = prc[1]
        c = prc[2]
        l0 = jnp.maximum(0, r - FO - y0)
        l1 = jnp.minimum(BAND, r + FO - y0)
        f0 = y0 + l0 + FO - r
        fb = jnp.minimum(f0, FH - L)
        pltpu.async_copy(filt_hbm.at[p * FH + fb + lane], fbuf, sem).wait()
        xbase = c - FO
        x0 = xbase + lane
        def row_body(l, carry2):
            fl = y0 + l + FO - r - fb
            lvec = jnp.full((L,), l, jnp.int32)
            for s in range(FW // L):
                x = x0 + s * L
                m = (x >= 0) & (x < W)
                xi = jnp.clip(x, 0, W - 1)
                v = fbuf[fl, s * L:(s + 1) * L]
                plsc.addupdate_scatter(band_v, [lvec, xi], v, mask=m)
            return carry2
        lax.fori_loop(l0, l1, row_body, 0)
        return carry
    lax.fori_loop(0, nhits, process, 0)

    pltpu.sync_copy(band_v, out_hbm.at[pl.ds(y0, BAND), :])


def kernel(phw_list, filters):
    phw_flat = phw_list.reshape(-1)
    filt2d = filters.reshape(NFILT * FH, FW)
    mesh = plsc.VectorSubcoreMesh(
        core_axis_name="c", subcore_axis_name="s", num_cores=NC, num_subcores=NS)
    run = functools.partial(
        pl.kernel,
        out_type=jax.ShapeDtypeStruct((H, W), jnp.float32),
        mesh=mesh,
        scratch_types=[
            pltpu.VMEM((NPART * 3 + L,), jnp.int32),
            pltpu.VMEM((NPART + L,), jnp.int32),
            pltpu.VMEM((L, FW), jnp.float32),
            pltpu.VMEM((BAND, W), jnp.float32),
            pltpu.SemaphoreType.DMA,
        ],
        compiler_params=pltpu.CompilerParams(
            needs_layout_passes=False, use_tc_tiling_on_sc=False),
    )(_body)
    return run(phw_flat, filt2d)


# speedup vs baseline: 73.0041x; 1.8584x over previous
import functools

import jax
import jax.numpy as jnp
from jax import lax
from jax.experimental import pallas as pl
from jax.experimental.pallas import tpu as pltpu
from jax.experimental.pallas import tpu_sc as plsc

NFILT = 512
FH = FW = 64
H = W = 512
NPART = 8192
NC, NS, L = 2, 16, 16
NW = NC * NS
BAND = H // NW
FO = FH // 2


def _body(phw_hbm, filt_hbm, out_hbm, phw_v, hits_v, fbuf, band_v, sems):
    wid = lax.axis_index("s") * NC + lax.axis_index("c")
    y0 = (wid * BAND).astype(jnp.int32)
    lane = lax.iota(jnp.int32, L)
    zv = jnp.zeros((L,), jnp.float32)

    pltpu.sync_copy(phw_hbm, phw_v.at[pl.ds(0, NPART * 3)])

    def zero_row(i, carry):
        for s in range(W // L):
            band_v[i, s * L:(s + 1) * L] = zv
        return carry
    lax.fori_loop(0, BAND, zero_row, 0)

    def scan_chunk(k, cnt):
        base = k * L
        r = plsc.load_gather(phw_v, [(base + lane) * 3 + 1])
        hit = (r >= y0 - (FO - 1)) & (r <= y0 + BAND + (FO - 1))
        plsc.store_compressed(hits_v.at[pl.ds(cnt, L)], base + lane, mask=hit)
        return cnt + jnp.sum(hit.astype(jnp.int32))
    nhits = lax.fori_loop(0, NPART // L, scan_chunk, jnp.int32(0))

    def part_geom(h):
        pid = hits_v[pl.ds(h, L)][0]
        prc = phw_v[pl.ds(pid * 3, L)]
        p = prc[0]
        r = prc[1]
        c = prc[2]
        l0 = jnp.maximum(0, r - FO - y0)
        l1 = jnp.minimum(BAND, r + FO - y0)
        f0 = y0 + l0 + FO - r
        fb = jnp.minimum(f0, FH - L)
        return p, r, c, l0, l1, fb

    def issue(h, slot):
        p, _, _, _, _, fb = part_geom(h)
        pltpu.async_copy(filt_hbm.at[p * FH + fb + lane], fbuf.at[slot],
                         sems.at[slot])

    @pl.when(nhits > 0)
    def _():
        issue(0, 0)

    def process(h, carry):
        slot = lax.rem(h, 2)
        @pl.when(h + 1 < nhits)
        def _():
            issue(h + 1, 1 - slot)
        _, r, c, l0, l1, fb = part_geom(h)
        x0 = c - FO + lane
        xi = []
        ms = []
        for s in range(FW // L):
            x = x0 + s * L
            ms.append((x >= 0) & (x < W))
            xi.append(jnp.clip(x, 0, W - 1))
        pltpu.make_async_copy(filt_hbm.at[lane], fbuf.at[slot],
                              sems.at[slot]).wait()
        foff = y0 + FO - r - fb
        def row_body(l, carry2):
            fl = foff + l
            lvec = jnp.full((L,), l, jnp.int32)
            for s in range(FW // L):
                v = fbuf[slot, fl, s * L:(s + 1) * L]
                plsc.addupdate_scatter(band_v, [lvec, xi[s]], v, mask=ms[s])
            return carry2
        lax.fori_loop(l0, l1, row_body, 0)
        return carry
    lax.fori_loop(0, nhits, process, 0)

    pltpu.sync_copy(band_v, out_hbm.at[pl.ds(y0, BAND), :])


def kernel(phw_list, filters):
    phw_flat = phw_list.reshape(-1)
    filt2d = filters.reshape(NFILT * FH, FW)
    mesh = plsc.VectorSubcoreMesh(
        core_axis_name="c", subcore_axis_name="s", num_cores=NC, num_subcores=NS)
    run = functools.partial(
        pl.kernel,
        out_type=jax.ShapeDtypeStruct((H, W), jnp.float32),
        mesh=mesh,
        scratch_types=[
            pltpu.VMEM((NPART * 3 + L,), jnp.int32),
            pltpu.VMEM((NPART + L,), jnp.int32),
            pltpu.VMEM((2, L, FW), jnp.float32),
            pltpu.VMEM((BAND, W), jnp.float32),
            pltpu.SemaphoreType.DMA((2,)),
        ],
        compiler_params=pltpu.CompilerParams(
            needs_layout_passes=False, use_tc_tiling_on_sc=False),
    )(_body)
    return run(phw_flat, filt2d)


# speedup vs baseline: 109.3612x; 1.4980x over previous
import functools

import jax
import jax.numpy as jnp
from jax import lax
from jax.experimental import pallas as pl
from jax.experimental.pallas import tpu as pltpu
from jax.experimental.pallas import tpu_sc as plsc

NFILT = 512
FH = FW = 64
H = W = 512
NPART = 8192
NC, NS, L = 2, 16, 16
NW = NC * NS
BAND = H // NW
FO = FH // 2
GB = 8


def _body(phw_hbm, filt_hbm, out_hbm, phw_v, hits_v, idxb, fbuf, band_f, sems):
    wid = lax.axis_index("s") * NC + lax.axis_index("c")
    y0 = (wid * BAND).astype(jnp.int32)
    lane = lax.iota(jnp.int32, L)
    zv = jnp.zeros((L,), jnp.float32)

    pltpu.sync_copy(phw_hbm, phw_v.at[pl.ds(0, NPART * 3)])

    def zero_chunk(i, carry):
        band_f[pl.ds(i * L, L)] = zv
        return carry
    lax.fori_loop(0, BAND * W // L, zero_chunk, 0)

    def scan_chunk(k, cnt):
        base = k * L
        r = plsc.load_gather(phw_v, [(base + lane) * 3 + 1])
        hit = (r >= y0 - (FO - 1)) & (r <= y0 + BAND + (FO - 1))
        plsc.store_compressed(hits_v.at[pl.ds(cnt, L)], base + lane, mask=hit)
        return cnt + jnp.sum(hit.astype(jnp.int32))
    nhits = lax.fori_loop(0, NPART // L, scan_chunk, jnp.int32(0))
    hits_v[pl.ds(nhits, L)] = jnp.zeros((L,), jnp.int32)

    def part_geom(h):
        pid = hits_v[pl.ds(h, L)][0]
        prc = phw_v[pl.ds(pid * 3, L)]
        p = prc[0]
        r = prc[1]
        c = prc[2]
        l0 = jnp.maximum(0, r - FO - y0)
        l1 = jnp.minimum(BAND, r + FO - y0)
        f0 = y0 + l0 + FO - r
        fb = jnp.minimum(f0, FH - L)
        return p, r, c, l0, l1, fb

    ngroups = lax.div(nhits + (GB - 1), jnp.int32(GB))

    def build_issue(g, slot):
        base = g * GB
        for j in range(GB):
            p, _, _, _, _, fb = part_geom(base + j)
            idxb[slot, j * L:(j + 1) * L] = p * FH + fb + lane
        pltpu.async_copy(filt_hbm.at[idxb.at[slot]], fbuf.at[slot],
                         sems.at[slot])

    @pl.when(ngroups > 0)
    def _():
        build_issue(0, 0)

    def process_group(g, carry):
        slot = lax.rem(g, 2)
        @pl.when(g + 1 < ngroups)
        def _():
            build_issue(g + 1, 1 - slot)
        pltpu.make_async_copy(filt_hbm.at[idxb.at[slot]], fbuf.at[slot],
                              sems.at[slot]).wait()
        for j in range(GB):
            h = g * GB + j
            @pl.when(h < nhits)
            def _():
                _, r, c, l0, l1, fb = part_geom(h)
                x0 = c - FO + lane
                xi = []
                ms = []
                for s in range(FW // L):
                    x = x0 + s * L
                    ms.append((x >= 0) & (x < W))
                    xi.append(jnp.clip(x, 0, W - 1))
                foff = y0 + FO - r - fb
                def row_body(l, carry2):
                    fl = foff + l + j * L
                    lw = l * W
                    for s in range(FW // L):
                        v = fbuf[slot, fl, s * L:(s + 1) * L]
                        plsc.addupdate_scatter(band_f, [xi[s] + lw], v,
                                               mask=ms[s])
                    return carry2
                lax.fori_loop(l0, l1, row_body, 0)
        return carry
    lax.fori_loop(0, ngroups, process_group, 0)

    pltpu.sync_copy(band_f, out_hbm.at[pl.ds(y0 * W, BAND * W)])


def kernel(phw_list, filters):
    phw_flat = phw_list.reshape(-1)
    filt2d = filters.reshape(NFILT * FH, FW)
    mesh = plsc.VectorSubcoreMesh(
        core_axis_name="c", subcore_axis_name="s", num_cores=NC, num_subcores=NS)
    run = functools.partial(
        pl.kernel,
        out_type=jax.ShapeDtypeStruct((H * W,), jnp.float32),
        mesh=mesh,
        scratch_types=[
            pltpu.VMEM((NPART * 3 + L,), jnp.int32),
            pltpu.VMEM((NPART + 2 * L,), jnp.int32),
            pltpu.VMEM((2, GB * L), jnp.int32),
            pltpu.VMEM((2, GB * L, FW), jnp.float32),
            pltpu.VMEM((BAND * W,), jnp.float32),
            pltpu.SemaphoreType.DMA((2,)),
        ],
        compiler_params=pltpu.CompilerParams(
            needs_layout_passes=False, use_tc_tiling_on_sc=False),
    )(_body)
    return run(phw_flat, filt2d).reshape(H, W)
